# baseline (speedup 1.0000x reference)
import functools

import jax
import jax.numpy as jnp
from jax import lax
from jax.experimental import pallas as pl
from jax.experimental.pallas import tpu as pltpu
from jax.experimental.pallas import tpu_sc as plsc

N = 10000
E = 320000
G = 20
R = 200
S = 5000
NP = 10240
RP = 256
SP = 5008
NC = 2
NS = 16
K = 128
NEG = float("-inf")

ROWS_A = 640
ROWS_SB = 320
CHA = 384
CHS = 104
CHB = 200
EPP = 157 * 2048

_mesh = plsc.VectorSubcoreMesh(core_axis_name="c", subcore_axis_name="s")


def _make_segown(n_chunks, rows_own, per_core_out):
    cap = n_chunks * K

    @functools.partial(
        pl.kernel,
        out_type=jax.ShapeDtypeStruct(
            (NC, NP, 128) if per_core_out else (NP, 128), jnp.float32),
        mesh=_mesh,
        scratch_types=[
            pltpu.VMEM((K,), jnp.int32),
            pltpu.VMEM((K,), jnp.int32),
            pltpu.VMEM((K, 128), jnp.float32),
            pltpu.VMEM((rows_own + 8, 128), jnp.float32),
            pltpu.SemaphoreType.DMA,
        ],
    )
    def segown(table, idx_in, dloc, out, ivec, dvec, rows, acc, sem):
        c = lax.axis_index("c")
        s = lax.axis_index("s")
        z = jnp.zeros((16,), jnp.float32)

        @pl.loop(0, rows_own + 8)
        def _(i):
            for j in range(8):
                acc[i, pl.ds(16 * j, 16)] = z

        @pl.loop(0, n_chunks)
        def _(kk):
            pltpu.sync_copy(idx_in.at[c, s, pl.ds(kk * K, K)], ivec)
            pltpu.sync_copy(dloc.at[c, s, pl.ds(kk * K, K)], dvec)
            pltpu.async_copy(table.at[ivec], rows, sem).wait()

            @pl.loop(0, K // 16)
            def _(jv):
                dv = dvec[pl.ds(16 * jv, 16)]
                for lane in range(16):
                    d = dv[lane]
                    j = 16 * jv + lane
                    for i in range(8):
                        sl = pl.ds(16 * i, 16)
                        acc[d, sl] = acc[d, sl] + rows[j, sl]

        if per_core_out:
            pltpu.sync_copy(acc.at[pl.ds(0, rows_own)],
                            out.at[c, pl.ds(s * rows_own, rows_own)])
        else:
            pltpu.sync_copy(acc.at[pl.ds(0, rows_own)],
                            out.at[pl.ds((c * NS + s) * rows_own, rows_own)])

    return segown


_segown_a = _make_segown(CHA, ROWS_A, True)
_segown_s = _make_segown(CHS, ROWS_SB, False)
_segown_b = _make_segown(CHB, ROWS_SB, False)


@functools.partial(
    pl.kernel,
    out_type=jax.ShapeDtypeStruct((2 * NP, 128), jnp.float32),
    mesh=_mesh,
    scratch_types=[
        pltpu.VMEM((64,), jnp.int32),
        pltpu.VMEM((64, 128), jnp.float32),
        pltpu.SemaphoreType.DMA,
    ],
)
def _prep(ent_e, ent_m, node_id, t1, ivec, rows, sem):
    c = lax.axis_index("c")
    s = lax.axis_index("s")
    wid = s * NC + c

    @pl.loop(0, NP // 32 // 64)
    def _(t):
        base = wid * (NP // 32) + t * 64
        pltpu.sync_copy(node_id.at[pl.ds(base, 64)], ivec)
        pltpu.async_copy(ent_e.at[ivec], rows, sem).wait()
        pltpu.sync_copy(rows, t1.at[pl.ds(base, 64)])
        pltpu.async_copy(ent_m.at[ivec], rows, sem).wait()
        pltpu.sync_copy(rows, t1.at[pl.ds(NP + base, 64)])


def _negrel_body(a, b, o):
    o[0:RP, :] = -a[...]
    o[RP:2 * RP, :] = -b[...]


def _tc1_body(a0, a1, sS, h0a, h0b, w1, wt1, w1l, h1):
    agg = (jnp.dot(a0[...], w1[0:128, :], preferred_element_type=jnp.float32)
           + jnp.dot(a1[...], w1[128:256, :], preferred_element_type=jnp.float32)
           + jnp.dot(sS[...], wt1[...], preferred_element_type=jnp.float32)
           + jnp.dot(h0a[...], w1l[0:128, :], preferred_element_type=jnp.float32)
           + jnp.dot(h0b[...], w1l[128:256, :], preferred_element_type=jnp.float32))
    h1[...] = jnp.maximum(agg, 0.0)


def _rel1_body(rn1, rn2, w1r, er1, ner1):
    v = jnp.maximum(
        jnp.dot(-rn1[...], w1r[0:128, :], preferred_element_type=jnp.float32)
        + jnp.dot(-rn2[...], w1r[128:256, :], preferred_element_type=jnp.float32),
        0.0)
    er1[...] = v
    ner1[...] = -v


def _tc2_body(bB, sS, h1, w2, wt2, w2l, h2):
    agg = (jnp.dot(bB[...], w2[...], preferred_element_type=jnp.float32)
           + jnp.dot(sS[...], wt2[...], preferred_element_type=jnp.float32)
           + jnp.dot(h1[...], w2l[...], preferred_element_type=jnp.float32))
    h2[...] = jnp.maximum(agg, 0.0)


def _rel2_body(er1, w2r, er2):
    er2[...] = jnp.maximum(
        jnp.dot(er1[...], w2r[...], preferred_element_type=jnp.float32), 0.0)


def _gn_body(h2, gid, gn):
    i = pl.program_id(0)

    @pl.when(i == 0)
    def _():
        gn[...] = jnp.full((32, 256), NEG, jnp.float32)

    x = h2[...]
    g = gid[...]
    rows = [jnp.max(jnp.where(g == k, x, NEG), axis=0, keepdims=True)
            for k in range(G)]
    rows.append(jnp.full((32 - G, 256), NEG, jnp.float32))
    gn[...] = jnp.maximum(gn[...], jnp.concatenate(rows, axis=0))


def _pres_body(et3, eg2, out):
    i = pl.program_id(0)

    @pl.when(i == 0)
    def _():
        out[...] = jnp.zeros((RP, 32), jnp.float32)

    etb = et3[...].reshape(1, 2048)
    ohr = jnp.where(lax.broadcasted_iota(jnp.int32, (RP, 2048), 0) == etb,
                    1.0, 0.0).astype(jnp.float32)
    egb = eg2[...]
    ohg = jnp.where(egb == lax.broadcasted_iota(jnp.int32, (2048, 32), 1),
                    1.0, 0.0).astype(jnp.float32)
    out[...] += jnp.dot(ohr, ohg, preferred_element_type=jnp.float32)


def _final_body(cnt, er2, gn, tidx, out):
    cv = cnt[...]
    er = er2[...]
    t = tidx[...]
    accn = jnp.full((80, 256), NEG, jnp.float32)
    acce = jnp.full((80, 256), NEG, jnp.float32)
    gnv = gn[...]
    for g in range(G):
        m = cv[:, g:g + 1] > 0.0
        gerow = jnp.max(jnp.where(m, er, NEG), axis=0, keepdims=True)
        sel = t == g
        accn = jnp.where(sel, gnv[g:g + 1, :], accn)
        acce = jnp.where(sel, gerow, acce)
    out[:, 0:256] = accn
    out[:, 256:512] = acce


def kernel(ent_embeds, ent_memory, rel_embeds, rel_memory, sent_table,
           W1, W1_loop, W1_rel, Wt1, W2, W2_loop, W2_rel, Wt2,
           node_id, src, dst, etype, sid, node_gid, edge_gid, time_idx):
    f32 = jnp.float32
    i32 = jnp.int32

    node_id_p = jnp.pad(node_id, (0, NP - N))
    rel_e_p = jnp.pad(rel_embeds, ((0, RP - R), (0, 0)))
    rel_m_p = jnp.pad(rel_memory, ((0, RP - R), (0, 0)))
    sent_p = jnp.pad(sent_table, ((0, SP - S), (0, 0)))

    key = dst // ROWS_SB
    order = jnp.argsort(key)
    ks = key[order]
    k6 = ks // 2
    ar = jnp.arange(E, dtype=i32)
    start32 = jnp.searchsorted(ks, jnp.arange(32, dtype=i32)).astype(i32)
    start16 = jnp.searchsorted(k6, jnp.arange(16, dtype=i32)).astype(i32)
    rank32 = ar - start32[ks]
    rank16 = ar - start16[k6]
    src_o, et_o, sid_o, dst_o = (src[order], etype[order], sid[order],
                                 dst[order])

    def routed(nb, cap_ent, pos_list, val_list, dfill, dpos, dval):
        outs = []
        for pos, val in zip(pos_list, val_list):
            outs.append(jnp.zeros(nb * cap_ent, i32).at[pos].set(val))
        dl = jnp.full(nb * cap_ent, dfill, i32)
        for p in dpos:
            dl = dl.at[p].set(dval)
        return outs, dl

    capA = CHA * K
    posA0 = k6 * capA + 2 * rank16
    posA1 = posA0 + 1
    dlocA_v = dst_o - k6 * ROWS_A
    (inA0,), dlA = routed(16, capA, [posA0], [src_o], ROWS_A, [posA0, posA1],
                          dlocA_v)
    inA0 = inA0.at[posA1].set(2 * NP + et_o)
    inA1 = (jnp.zeros(16 * capA, i32).at[posA0].set(NP + src_o)
            .at[posA1].set(2 * NP + RP + et_o))
    inA = jnp.stack([inA0, inA1]).reshape(2, 16, capA)
    dlA2 = jnp.stack([dlA, dlA]).reshape(2, 16, capA)

    capS = CHS * K
    posS = ks * capS + rank32
    dlocS_v = dst_o - ks * ROWS_SB
    inS = jnp.zeros(32 * capS, i32).at[posS].set(sid_o).reshape(2, 16, capS)
    dlS = (jnp.full(32 * capS, ROWS_SB, i32).at[posS].set(dlocS_v)
           .reshape(2, 16, capS))

    capB = CHB * K
    posB0 = ks * capB + 2 * rank32
    posB1 = posB0 + 1
    inB = (jnp.zeros(32 * capB, i32).at[posB0].set(src_o)
           .at[posB1].set(NP + et_o)).reshape(2, 16, capB)
    dlB = (jnp.full(32 * capB, ROWS_SB, i32).at[posB0].set(dlocS_v)
           .at[posB1].set(dlocS_v).reshape(2, 16, capB))

    gid_p = jnp.pad(node_gid, (0, NP - N), constant_values=G).reshape(NP, 1)
    tflat = time_idx.reshape(-1, 1).astype(i32)
    eg_p = jnp.pad(edge_gid, (0, EPP - E), constant_values=31).reshape(EPP, 1)
    et_p = jnp.pad(etype, (0, EPP - E)).reshape(EPP // 2048, 1, 2048)

    negrel = pl.pallas_call(
        _negrel_body,
        out_shape=jax.ShapeDtypeStruct((2 * RP, 128), f32),
    )(rel_e_p, rel_m_p)
    t1 = jnp.concatenate([_prep(ent_embeds, ent_memory, node_id_p), negrel], 0)
    acc_a = _segown_a(t1, inA, dlA2)
    acc_s = _segown_s(sent_p, inS, dlS)

    nblk = NP // 256
    bs = lambda idx_map: pl.BlockSpec((256, 128), idx_map)
    full = lambda shp: pl.BlockSpec(shp, lambda i: (0, 0))
    h1 = pl.pallas_call(
        _tc1_body,
        grid=(nblk,),
        in_specs=[bs(lambda i: (i, 0)), bs(lambda i: (i, 0)),
                  bs(lambda i: (i, 0)),
                  bs(lambda i: (i, 0)), bs(lambda i: (NP // 256 + i, 0)),
                  full((256, 128)), full((128, 128)), full((256, 128))],
        out_specs=bs(lambda i: (i, 0)),
        out_shape=jax.ShapeDtypeStruct((NP, 128), f32),
    )(acc_a[0], acc_a[1], acc_s, t1, t1, W1, Wt1, W1_loop)

    er1, ner1 = pl.pallas_call(
        _rel1_body,
        out_shape=[jax.ShapeDtypeStruct((RP, 128), f32),
                   jax.ShapeDtypeStruct((RP, 128), f32)],
    )(t1[2 * NP:2 * NP + RP], t1[2 * NP + RP:], W1_rel)

    t2 = jnp.concatenate([h1, ner1], axis=0)
    acc_b = _segown_b(t2, inB, dlB)

    bs2 = lambda: pl.BlockSpec((256, 256), lambda i: (i, 0))
    h2 = pl.pallas_call(
        _tc2_body,
        grid=(nblk,),
        in_specs=[bs(lambda i: (i, 0)), bs(lambda i: (i, 0)),
                  bs(lambda i: (i, 0)),
                  full((128, 256)), full((128, 256)), full((128, 256))],
        out_specs=bs2(),
        out_shape=jax.ShapeDtypeStruct((NP, 256), f32),
    )(acc_b, acc_s, h1, W2, Wt2, W2_loop)

    er2 = pl.pallas_call(
        _rel2_body,
        out_shape=jax.ShapeDtypeStruct((RP, 256), f32),
    )(er1, W2_rel)

    gn = pl.pallas_call(
        _gn_body,
        grid=(nblk,),
        in_specs=[bs2(), pl.BlockSpec((256, 1), lambda i: (i, 0))],
        out_specs=pl.BlockSpec((32, 256), lambda i: (0, 0)),
        out_shape=jax.ShapeDtypeStruct((32, 256), f32),
    )(h2, gid_p)

    cnt = pl.pallas_call(
        _pres_body,
        grid=(EPP // 2048,),
        in_specs=[pl.BlockSpec((1, 1, 2048), lambda i: (i, 0, 0)),
                  pl.BlockSpec((2048, 1), lambda i: (i, 0))],
        out_specs=pl.BlockSpec((RP, 32), lambda i: (0, 0)),
        out_shape=jax.ShapeDtypeStruct((RP, 32), f32),
    )(et_p, eg_p)

    seq = pl.pallas_call(
        _final_body,
        out_shape=jax.ShapeDtypeStruct((80, 512), f32),
    )(cnt, er2, gn, tflat)

    return seq.reshape(time_idx.shape[0], time_idx.shape[1], 512)
